# baseline (speedup 1.0000x reference)
import functools

import jax
import jax.numpy as jnp
from jax import lax
from jax.experimental import pallas as pl
from jax.experimental.pallas import tpu as pltpu
from jax.experimental.pallas import tpu_sc as plsc

_E = 16
_K = 2
_T = 256
_MAXT = 32
_IH = 1


def _routing_body(ti_ref, pos_ref, texp_ref, trows_ref):
    p = ti_ref.shape[0]
    ev = ti_ref[:, :]
    e_iota = lax.broadcasted_iota(jnp.int32, (1, _E), 1)
    occ = (ev == e_iota).astype(jnp.int32)
    csum = occ
    sh = 1
    while sh < p:
        z = jnp.zeros((sh, _E), jnp.int32)
        csum = csum + jnp.concatenate([z, csum[: p - sh, :]], axis=0)
        sh *= 2
    counts = csum[p - 1 : p, :]
    tiles = (counts + (_T - 1)) // _T
    r_i = lax.broadcasted_iota(jnp.int32, (_E, _E), 0)
    c_i = lax.broadcasted_iota(jnp.int32, (_E, _E), 1)
    strict_lt = (r_i < c_i).astype(jnp.float32)
    toff = jax.lax.dot_general(
        tiles.astype(jnp.float32), strict_lt,
        (((1,), (0,)), ((), ())), preferred_element_type=jnp.float32,
    ).astype(jnp.int32)
    base = toff * _T
    rank = jnp.sum(occ * csum, axis=1, keepdims=True) - 1
    pos_ref[:, :] = jnp.sum(occ * base, axis=1, keepdims=True) + rank

    t_col = lax.broadcasted_iota(jnp.int32, (_MAXT, _E), 0)
    m = (t_col >= toff) & (t_col < toff + tiles)
    mi = m.astype(jnp.int32)
    texp_raw = jnp.sum(mi * e_iota, axis=1, keepdims=True)
    start = jnp.sum(mi * toff, axis=1, keepdims=True)
    cnt = jnp.sum(mi * counts, axis=1, keepdims=True)
    t1 = lax.broadcasted_iota(jnp.int32, (_MAXT, 1), 0)
    trows_ref[:, :] = jnp.clip(cnt - (t1 - start) * _T, 0, _T)
    total = jnp.sum(tiles)
    last_e = jnp.sum(jnp.where(t1 == total - 1, texp_raw, 0))
    texp_ref[:, :] = jnp.where(t1 < total, texp_raw, last_e)


def _routing(ti_flat):
    p = ti_flat.shape[0]
    return pl.pallas_call(
        _routing_body,
        grid=(1,),
        in_specs=[pl.BlockSpec((p, 1), lambda i: (0, 0))],
        out_specs=[
            pl.BlockSpec((p, 1), lambda i: (0, 0)),
            pl.BlockSpec((_MAXT, 1), lambda i: (0, 0)),
            pl.BlockSpec((_MAXT, 1), lambda i: (0, 0)),
        ],
        out_shape=[
            jax.ShapeDtypeStruct((p, 1), jnp.int32),
            jax.ShapeDtypeStruct((_MAXT, 1), jnp.int32),
            jax.ShapeDtypeStruct((_MAXT, 1), jnp.int32),
        ],
    )(ti_flat)


def _sc_dispatch(hidden, pos, n_slots):
    n, h = hidden.shape
    p = pos.shape[0]
    info = plsc.get_sparse_core_info()
    nw = info.num_cores * info.num_subcores
    per_w = p // nw
    ch = 16
    nch = per_w // ch
    mesh = plsc.VectorSubcoreMesh(core_axis_name="c", subcore_axis_name="s")

    @functools.partial(
        pl.kernel,
        mesh=mesh,
        out_type=jax.ShapeDtypeStruct((n_slots, h), jnp.float32),
        scratch_types=[
            pltpu.VMEM((ch,), jnp.int32),
            pltpu.VMEM((ch,), jnp.int32),
            pltpu.VMEM((ch,), jnp.int32),
            pltpu.VMEM((ch,), jnp.int32),
            pltpu.VMEM((ch, h), jnp.float32),
            pltpu.VMEM((ch, h), jnp.float32),
            pltpu.SemaphoreType.DMA,
            pltpu.SemaphoreType.DMA,
            pltpu.SemaphoreType.DMA,
            pltpu.SemaphoreType.DMA,
        ],
    )
    def k(hid_hbm, pos_hbm, xs_hbm,
          tokv0, tokv1, posv0, posv1, rows0, rows1, gs0, gs1, ss0, ss1):
        wid = lax.axis_index("s") * info.num_cores + lax.axis_index("c")
        base = wid * per_w
        iot = lax.iota(jnp.int32, 16)
        tokv = (tokv0, tokv1)
        posv = (posv0, posv1)
        rows = (rows0, rows1)
        gs = (gs0, gs1)
        ss = (ss0, ss1)

        def start_gather(c):
            b = c & 1
            off = base + c * ch
            for j in range(ch // 16):
                tokv[b][pl.ds(j * 16, 16)] = lax.shift_right_logical(
                    off + j * 16 + iot, 1)
            return pltpu.async_copy(hid_hbm.at[tokv[b]], rows[b], gs[b])

        hg = [None] * nch
        hs = [None] * nch
        hg[0] = start_gather(0)
        for c in range(nch):
            b = c & 1
            if c + 1 < nch:
                if c >= 1:
                    hs[c - 1].wait()
                hg[c + 1] = start_gather(c + 1)
            hg[c].wait()
            pltpu.sync_copy(pos_hbm.at[pl.ds(base + c * ch, ch)], posv[b])
            hs[c] = pltpu.async_copy(rows[b], xs_hbm.at[posv[b]], ss[b])
        if nch >= 2:
            hs[nch - 2].wait()
        hs[nch - 1].wait()

    return k(hidden, pos)


def _sc_gather(table, idx):
    v, h = table.shape
    p = idx.shape[0]
    info = plsc.get_sparse_core_info()
    nw = info.num_cores * info.num_subcores
    per_w = p // nw
    ch = 16
    nch = per_w // ch
    mesh = plsc.VectorSubcoreMesh(core_axis_name="c", subcore_axis_name="s")

    @functools.partial(
        pl.kernel,
        mesh=mesh,
        out_type=jax.ShapeDtypeStruct((p, h), jnp.float32),
        scratch_types=[
            pltpu.VMEM((ch,), jnp.int32),
            pltpu.VMEM((ch,), jnp.int32),
            pltpu.VMEM((ch, h), jnp.float32),
            pltpu.VMEM((ch, h), jnp.float32),
            pltpu.SemaphoreType.DMA,
            pltpu.SemaphoreType.DMA,
            pltpu.SemaphoreType.DMA,
            pltpu.SemaphoreType.DMA,
        ],
    )
    def k(tab_hbm, idx_hbm, out_hbm,
          idxv0, idxv1, rows0, rows1, gs0, gs1, ws0, ws1):
        wid = lax.axis_index("s") * info.num_cores + lax.axis_index("c")
        base = wid * per_w
        idxv = (idxv0, idxv1)
        rows = (rows0, rows1)
        gs = (gs0, gs1)
        ws = (ws0, ws1)

        def start_gather(c):
            b = c & 1
            pltpu.sync_copy(idx_hbm.at[pl.ds(base + c * ch, ch)], idxv[b])
            return pltpu.async_copy(tab_hbm.at[idxv[b]], rows[b], gs[b])

        hg = [None] * nch
        hw = [None] * nch
        hg[0] = start_gather(0)
        for c in range(nch):
            b = c & 1
            if c + 1 < nch:
                if c >= 1:
                    hw[c - 1].wait()
                hg[c + 1] = start_gather(c + 1)
            hg[c].wait()
            hw[c] = pltpu.async_copy(
                rows[b], out_hbm.at[pl.ds(base + c * ch, ch)], ws[b])
        if nch >= 2:
            hw[nch - 2].wait()
        hw[nch - 1].wait()

    return k(table, idx)


def _gemm_body(texp_ref, trows_ref, xs_ref, g_ref, u_ref, d_ref, o_ref):
    t = pl.program_id(0)
    ih = pl.program_id(1)

    @pl.when(trows_ref[t] > 0)
    def _go():
        x = xs_ref[...]
        dims = (((1,), (1,)), ((), ()))
        g = jax.lax.dot_general(x, g_ref[0], dims, preferred_element_type=jnp.float32)
        u = jax.lax.dot_general(x, u_ref[0], dims, preferred_element_type=jnp.float32)
        h = (g * (1.0 / (1.0 + jnp.exp(-g)))) * u
        part = jax.lax.dot_general(h, d_ref[0], dims, preferred_element_type=jnp.float32)

        @pl.when(ih == 0)
        def _w0():
            o_ref[...] = part

        @pl.when(ih > 0)
        def _w1():
            o_ref[...] += part


def _grouped_gemm(xs, texp, trows, gate, up, down):
    s, hid = xs.shape
    e, inter, _ = gate.shape
    ic = inter // _IH
    grid_spec = pltpu.PrefetchScalarGridSpec(
        num_scalar_prefetch=2,
        grid=(_MAXT, _IH),
        in_specs=[
            pl.BlockSpec((_T, hid),
                         lambda t, i, te, tr: (jnp.where(tr[t] > 0, t, _MAXT - 1), 0)),
            pl.BlockSpec((1, ic, hid), lambda t, i, te, tr: (te[t], i, 0)),
            pl.BlockSpec((1, ic, hid), lambda t, i, te, tr: (te[t], i, 0)),
            pl.BlockSpec((1, hid, ic), lambda t, i, te, tr: (te[t], 0, i)),
        ],
        out_specs=pl.BlockSpec(
            (_T, hid), lambda t, i, te, tr: (jnp.where(tr[t] > 0, t, _MAXT - 1), 0)),
    )
    return pl.pallas_call(
        _gemm_body,
        grid_spec=grid_spec,
        out_shape=jax.ShapeDtypeStruct((s, hid), jnp.float32),
        compiler_params=pltpu.CompilerParams(
            vmem_limit_bytes=100 * 1024 * 1024),
    )(texp, trows, xs, gate, up, down)


def _pairsum_body(xg_ref, tw_ref, o_ref):
    w0 = tw_ref[:, 0:1]
    w1 = tw_ref[:, 1:2]
    o_ref[...] = xg_ref[:, 0, :] * w0 + xg_ref[:, 1, :] * w1


def _pairsum(xg3, tw):
    n, k, hid = xg3.shape
    blk = 256
    return pl.pallas_call(
        _pairsum_body,
        grid=(n // blk,),
        in_specs=[
            pl.BlockSpec((blk, k, hid), lambda i: (i, 0, 0)),
            pl.BlockSpec((blk, k), lambda i: (i, 0)),
        ],
        out_specs=pl.BlockSpec((blk, hid), lambda i: (i, 0)),
        out_shape=jax.ShapeDtypeStruct((n, hid), jnp.float32),
    )(xg3, tw)


def kernel(hidden_states, topk_idx, topk_weight, gate_weight, up_weight, down_weight):
    bsz, seq_len, hidden = hidden_states.shape
    n = bsz * seq_len
    x = hidden_states.reshape(n, hidden)
    ti = topk_idx.astype(jnp.int32).reshape(n * _K, 1)
    n_slots = _MAXT * _T

    pos, texp, trows = _routing(ti)
    pos1 = pos.reshape(n * _K)
    xs = _sc_dispatch(x, pos1, n_slots)
    osorted = _grouped_gemm(xs, texp.reshape(_MAXT), trows.reshape(_MAXT),
                            gate_weight, up_weight, down_weight)
    half = (n * _K) // 2
    xg1 = _sc_gather(osorted, pos1[:half])
    xg2 = _sc_gather(osorted, pos1[half:])
    o1 = _pairsum(xg1.reshape(n // 2, _K, hidden), topk_weight[: n // 2])
    o2 = _pairsum(xg2.reshape(n // 2, _K, hidden), topk_weight[n // 2 :])
    out = jnp.concatenate([o1, o2], axis=0)
    return out.reshape(bsz, seq_len, hidden)


# speedup vs baseline: 1.0342x; 1.0342x over previous
import functools

import jax
import jax.numpy as jnp
from jax import lax
from jax.experimental import pallas as pl
from jax.experimental.pallas import tpu as pltpu
from jax.experimental.pallas import tpu_sc as plsc

_E = 16
_K = 2
_T = 256
_MAXT = 32
_IH = 1


def _routing_body(ti_ref, pos_ref, texp_ref, trows_ref):
    p = ti_ref.shape[0]
    ev = ti_ref[:, :]
    e_iota = lax.broadcasted_iota(jnp.int32, (1, _E), 1)
    occ = (ev == e_iota).astype(jnp.int32)
    csum = occ
    sh = 1
    while sh < p:
        z = jnp.zeros((sh, _E), jnp.int32)
        csum = csum + jnp.concatenate([z, csum[: p - sh, :]], axis=0)
        sh *= 2
    counts = csum[p - 1 : p, :]
    tiles = (counts + (_T - 1)) // _T
    r_i = lax.broadcasted_iota(jnp.int32, (_E, _E), 0)
    c_i = lax.broadcasted_iota(jnp.int32, (_E, _E), 1)
    strict_lt = (r_i < c_i).astype(jnp.float32)
    toff = jax.lax.dot_general(
        tiles.astype(jnp.float32), strict_lt,
        (((1,), (0,)), ((), ())), preferred_element_type=jnp.float32,
    ).astype(jnp.int32)
    base = toff * _T
    rank = jnp.sum(occ * csum, axis=1, keepdims=True) - 1
    pos_ref[:, :] = jnp.sum(occ * base, axis=1, keepdims=True) + rank

    t_col = lax.broadcasted_iota(jnp.int32, (_MAXT, _E), 0)
    m = (t_col >= toff) & (t_col < toff + tiles)
    mi = m.astype(jnp.int32)
    texp_raw = jnp.sum(mi * e_iota, axis=1, keepdims=True)
    start = jnp.sum(mi * toff, axis=1, keepdims=True)
    cnt = jnp.sum(mi * counts, axis=1, keepdims=True)
    t1 = lax.broadcasted_iota(jnp.int32, (_MAXT, 1), 0)
    trows_ref[:, :] = jnp.clip(cnt - (t1 - start) * _T, 0, _T)
    total = jnp.sum(tiles)
    last_e = jnp.sum(jnp.where(t1 == total - 1, texp_raw, 0))
    texp_ref[:, :] = jnp.where(t1 < total, texp_raw, last_e)


def _routing(ti_flat):
    p = ti_flat.shape[0]
    return pl.pallas_call(
        _routing_body,
        grid=(1,),
        in_specs=[pl.BlockSpec((p, 1), lambda i: (0, 0))],
        out_specs=[
            pl.BlockSpec((p, 1), lambda i: (0, 0)),
            pl.BlockSpec((_MAXT, 1), lambda i: (0, 0)),
            pl.BlockSpec((_MAXT, 1), lambda i: (0, 0)),
        ],
        out_shape=[
            jax.ShapeDtypeStruct((p, 1), jnp.int32),
            jax.ShapeDtypeStruct((_MAXT, 1), jnp.int32),
            jax.ShapeDtypeStruct((_MAXT, 1), jnp.int32),
        ],
    )(ti_flat)


def _sc_dispatch(hidden, pos, n_slots):
    n, h = hidden.shape
    p = pos.shape[0]
    info = plsc.get_sparse_core_info()
    nw = info.num_cores * info.num_subcores
    per_w = p // nw
    ch = 16
    nch = per_w // ch
    mesh = plsc.VectorSubcoreMesh(core_axis_name="c", subcore_axis_name="s")

    @functools.partial(
        pl.kernel,
        mesh=mesh,
        out_type=jax.ShapeDtypeStruct((n_slots, h), jnp.float32),
        scratch_types=[
            pltpu.VMEM((ch,), jnp.int32),
            pltpu.VMEM((ch,), jnp.int32),
            pltpu.VMEM((ch,), jnp.int32),
            pltpu.VMEM((ch,), jnp.int32),
            pltpu.VMEM((ch, h), jnp.float32),
            pltpu.VMEM((ch, h), jnp.float32),
            pltpu.SemaphoreType.DMA,
            pltpu.SemaphoreType.DMA,
            pltpu.SemaphoreType.DMA,
            pltpu.SemaphoreType.DMA,
        ],
    )
    def k(hid_hbm, pos_hbm, xs_hbm,
          tokv0, tokv1, posv0, posv1, rows0, rows1, gs0, gs1, ss0, ss1):
        wid = lax.axis_index("s") * info.num_cores + lax.axis_index("c")
        base = wid * per_w
        iot = lax.iota(jnp.int32, 16)
        tokv = (tokv0, tokv1)
        posv = (posv0, posv1)
        rows = (rows0, rows1)
        gs = (gs0, gs1)
        ss = (ss0, ss1)

        def start_gather(c):
            b = c & 1
            off = base + c * ch
            for j in range(ch // 16):
                tokv[b][pl.ds(j * 16, 16)] = lax.shift_right_logical(
                    off + j * 16 + iot, 1)
            return pltpu.async_copy(hid_hbm.at[tokv[b]], rows[b], gs[b])

        hg = [None] * nch
        hs = [None] * nch
        hg[0] = start_gather(0)
        for c in range(nch):
            b = c & 1
            if c + 1 < nch:
                if c >= 1:
                    hs[c - 1].wait()
                hg[c + 1] = start_gather(c + 1)
            hg[c].wait()
            pltpu.sync_copy(pos_hbm.at[pl.ds(base + c * ch, ch)], posv[b])
            hs[c] = pltpu.async_copy(rows[b], xs_hbm.at[posv[b]], ss[b])
        if nch >= 2:
            hs[nch - 2].wait()
        hs[nch - 1].wait()

    return k(hidden, pos)


def _sc_gather(table, idx):
    v, h = table.shape
    p = idx.shape[0]
    info = plsc.get_sparse_core_info()
    nw = info.num_cores * info.num_subcores
    per_w = p // nw
    ch = 16
    nch = per_w // ch
    mesh = plsc.VectorSubcoreMesh(core_axis_name="c", subcore_axis_name="s")

    @functools.partial(
        pl.kernel,
        mesh=mesh,
        out_type=jax.ShapeDtypeStruct((p, h), jnp.float32),
        scratch_types=[
            pltpu.VMEM((ch,), jnp.int32),
            pltpu.VMEM((ch,), jnp.int32),
            pltpu.VMEM((ch, h), jnp.float32),
            pltpu.VMEM((ch, h), jnp.float32),
            pltpu.SemaphoreType.DMA,
            pltpu.SemaphoreType.DMA,
            pltpu.SemaphoreType.DMA,
            pltpu.SemaphoreType.DMA,
        ],
    )
    def k(tab_hbm, idx_hbm, out_hbm,
          idxv0, idxv1, rows0, rows1, gs0, gs1, ws0, ws1):
        wid = lax.axis_index("s") * info.num_cores + lax.axis_index("c")
        base = wid * per_w
        idxv = (idxv0, idxv1)
        rows = (rows0, rows1)
        gs = (gs0, gs1)
        ws = (ws0, ws1)

        def start_gather(c):
            b = c & 1
            pltpu.sync_copy(idx_hbm.at[pl.ds(base + c * ch, ch)], idxv[b])
            return pltpu.async_copy(tab_hbm.at[idxv[b]], rows[b], gs[b])

        hg = [None] * nch
        hw = [None] * nch
        hg[0] = start_gather(0)
        for c in range(nch):
            b = c & 1
            if c + 1 < nch:
                if c >= 1:
                    hw[c - 1].wait()
                hg[c + 1] = start_gather(c + 1)
            hg[c].wait()
            hw[c] = pltpu.async_copy(
                rows[b], out_hbm.at[pl.ds(base + c * ch, ch)], ws[b])
        if nch >= 2:
            hw[nch - 2].wait()
        hw[nch - 1].wait()

    return k(table, idx)


def _gemm_body(texp_ref, trows_ref, xs_ref, g_ref, u_ref, d_ref, o_ref):
    t = pl.program_id(0)
    ih = pl.program_id(1)

    @pl.when(trows_ref[t] > 0)
    def _go():
        x = xs_ref[...]
        dims = (((1,), (1,)), ((), ()))
        g = jax.lax.dot_general(x, g_ref[0], dims, preferred_element_type=jnp.float32)
        u = jax.lax.dot_general(x, u_ref[0], dims, preferred_element_type=jnp.float32)
        h = (g * (1.0 / (1.0 + jnp.exp(-g)))) * u
        part = jax.lax.dot_general(h, d_ref[0], dims, preferred_element_type=jnp.float32)

        @pl.when(ih == 0)
        def _w0():
            o_ref[...] = part

        @pl.when(ih > 0)
        def _w1():
            o_ref[...] += part


def _grouped_gemm(xs, texp, trows, gate, up, down):
    s, hid = xs.shape
    e, inter, _ = gate.shape
    ic = inter // _IH
    grid_spec = pltpu.PrefetchScalarGridSpec(
        num_scalar_prefetch=2,
        grid=(_MAXT, _IH),
        in_specs=[
            pl.BlockSpec((_T, hid),
                         lambda t, i, te, tr: (jnp.where(tr[t] > 0, t, _MAXT - 1), 0)),
            pl.BlockSpec((1, ic, hid), lambda t, i, te, tr: (te[t], i, 0)),
            pl.BlockSpec((1, ic, hid), lambda t, i, te, tr: (te[t], i, 0)),
            pl.BlockSpec((1, hid, ic), lambda t, i, te, tr: (te[t], 0, i)),
        ],
        out_specs=pl.BlockSpec(
            (_T, hid), lambda t, i, te, tr: (jnp.where(tr[t] > 0, t, _MAXT - 1), 0)),
    )
    return pl.pallas_call(
        _gemm_body,
        grid_spec=grid_spec,
        out_shape=jax.ShapeDtypeStruct((s, hid), jnp.float32),
        compiler_params=pltpu.CompilerParams(
            vmem_limit_bytes=100 * 1024 * 1024),
    )(texp, trows, xs, gate, up, down)


def _pairsum_body(xg_ref, tw_ref, o_ref):
    w0 = tw_ref[:, 0:1]
    w1 = tw_ref[:, 1:2]
    o_ref[...] = xg_ref[:, 0, :] * w0 + xg_ref[:, 1, :] * w1


def _pairsum(xg3, tw):
    n, k, hid = xg3.shape
    blk = 256
    return pl.pallas_call(
        _pairsum_body,
        grid=(n // blk,),
        in_specs=[
            pl.BlockSpec((blk, k, hid), lambda i: (i, 0, 0)),
            pl.BlockSpec((blk, k), lambda i: (i, 0)),
        ],
        out_specs=pl.BlockSpec((blk, hid), lambda i: (i, 0)),
        out_shape=jax.ShapeDtypeStruct((n, hid), jnp.float32),
    )(xg3, tw)


def kernel(hidden_states, topk_idx, topk_weight, gate_weight, up_weight, down_weight):
    bsz, seq_len, hidden = hidden_states.shape
    n = bsz * seq_len
    x = hidden_states.reshape(n, hidden)
    ti = topk_idx.astype(jnp.int32).reshape(n * _K, 1)
    n_slots = _MAXT * _T

    pos, texp, trows = _routing(ti)
    pos1 = pos.reshape(n * _K)
    xs = _sc_dispatch(x, pos1, n_slots)
    osorted = _grouped_gemm(xs, texp.reshape(_MAXT), trows.reshape(_MAXT),
                            gate_weight, up_weight, down_weight)
    xg = _sc_gather(osorted, pos1)
    out = _pairsum(xg.reshape(n, _K, hidden), topk_weight)
    return out.reshape(bsz, seq_len, hidden)
